# baseline (speedup 1.0000x reference)
import jax
import jax.numpy as jnp
from jax.experimental import pallas as pl
from jax.experimental.pallas import tpu as pltpu


def _fused_body(x_ref, pe_ref, batch_ref, wex_ref, wepe_ref, be_ref,
                pw1_ref, pb1_ref, pw2_ref, pb2_ref,
                rw1_ref, rb1_ref, rw2_ref, rb2_ref,
                out_ref, acc_ref):
    i = pl.program_id(0)
    nblk = pl.num_programs(0)

    @pl.when(i == 0)
    def _init():
        acc_ref[:] = jnp.zeros_like(acc_ref)

    h = jnp.dot(x_ref[:], wex_ref[:], preferred_element_type=jnp.float32)
    h += jnp.dot(pe_ref[:], wepe_ref[:], preferred_element_type=jnp.float32)
    h = (h + be_ref[:]) * 16.0

    t = jax.nn.silu(
        jnp.dot(h, pw1_ref[:], preferred_element_type=jnp.float32) + pb1_ref[:])
    t = jnp.dot(t, pw2_ref[:], preferred_element_type=jnp.float32) + pb2_ref[:]

    g = acc_ref.shape[0]
    onehot = (batch_ref[:] == jax.lax.broadcasted_iota(
        jnp.int32, (t.shape[0], g), 1)).astype(jnp.float32)
    acc_ref[:] += jax.lax.dot_general(
        onehot, t, (((0,), (0,)), ((), ())),
        preferred_element_type=jnp.float32)

    @pl.when(i == nblk - 1)
    def _readout():
        p = acc_ref[:]
        r = jax.nn.silu(
            jnp.dot(p, rw1_ref[:], preferred_element_type=jnp.float32)
            + rb1_ref[:])
        out_ref[:] = (
            jnp.dot(r, rw2_ref[:], preferred_element_type=jnp.float32)
            + rb2_ref[:])


def kernel(x, pos, pe, edge_index, batch, W_embed, b_embed, msg_W1, msg_b1,
           msg_W2, msg_b2, pre_W1, pre_b1, pre_W2, pre_b2, ro_W1, ro_b1,
           ro_W2, ro_b2):
    n, d_in = x.shape
    ped = pe.shape[1]
    h_dim = W_embed.shape[1]
    g = 64
    out_dim = ro_W2.shape[1]

    blk = 1000
    assert n % blk == 0
    grid = (n // blk,)

    wex = W_embed[:d_in]
    wepe = W_embed[d_in:]
    batch2 = batch.reshape(n, 1)

    full = lambda shape: pl.BlockSpec(shape, lambda i: (0, 0))
    out = pl.pallas_call(
        _fused_body,
        grid=grid,
        in_specs=[
            pl.BlockSpec((blk, d_in), lambda i: (i, 0)),
            pl.BlockSpec((blk, ped), lambda i: (i, 0)),
            pl.BlockSpec((blk, 1), lambda i: (i, 0)),
            full((d_in, h_dim)),
            full((ped, h_dim)),
            full((1, h_dim)),
            full((h_dim, h_dim)),
            full((1, h_dim)),
            full((h_dim, h_dim)),
            full((1, h_dim)),
            full((h_dim, h_dim)),
            full((1, h_dim)),
            full((h_dim, out_dim)),
            full((1, out_dim)),
        ],
        out_specs=pl.BlockSpec((g, out_dim), lambda i: (0, 0)),
        out_shape=jax.ShapeDtypeStruct((g, out_dim), jnp.float32),
        scratch_shapes=[pltpu.VMEM((g, h_dim), jnp.float32)],
        compiler_params=pltpu.CompilerParams(
            dimension_semantics=("arbitrary",)),
    )(x, pe, batch2, wex, wepe, b_embed.reshape(1, h_dim),
      pre_W1, pre_b1.reshape(1, h_dim), pre_W2, pre_b2.reshape(1, h_dim),
      ro_W1, ro_b1.reshape(1, h_dim), ro_W2, ro_b2.reshape(1, out_dim))
    return jnp.squeeze(out)


# speedup vs baseline: 1.1022x; 1.1022x over previous
import jax
import jax.numpy as jnp
from jax.experimental import pallas as pl
from jax.experimental.pallas import tpu as pltpu


def _fused_body(x_ref, pe_ref, batch_ref, wex_ref, wepe_ref, be_ref,
                pw1_ref, pb1_ref, pw2_ref, pb2_ref,
                rw1_ref, rb1_ref, rw2_ref, rb2_ref,
                out_ref, acc_ref):
    i = pl.program_id(0)
    nblk = pl.num_programs(0)

    @pl.when(i == 0)
    def _init():
        acc_ref[:] = jnp.zeros_like(acc_ref)

    h = jnp.dot(x_ref[:], wex_ref[:], preferred_element_type=jnp.float32)
    h += jnp.dot(pe_ref[:], wepe_ref[:], preferred_element_type=jnp.float32)
    h = (h + be_ref[:]) * 16.0

    t = jax.nn.silu(
        jnp.dot(h, pw1_ref[:], preferred_element_type=jnp.float32) + pb1_ref[:])
    t = jnp.dot(t, pw2_ref[:], preferred_element_type=jnp.float32) + pb2_ref[:]

    g = acc_ref.shape[0]
    onehot = (batch_ref[:] == jax.lax.broadcasted_iota(
        jnp.int32, (t.shape[0], g), 1)).astype(jnp.float32)
    acc_ref[:] += jax.lax.dot_general(
        onehot, t, (((0,), (0,)), ((), ())),
        preferred_element_type=jnp.float32)

    @pl.when(i == nblk - 1)
    def _readout():
        p = acc_ref[:]
        r = jax.nn.silu(
            jnp.dot(p, rw1_ref[:], preferred_element_type=jnp.float32)
            + rb1_ref[:])
        out_ref[:] = (
            jnp.dot(r, rw2_ref[:], preferred_element_type=jnp.float32)
            + rb2_ref[:])


def kernel(x, pos, pe, edge_index, batch, W_embed, b_embed, msg_W1, msg_b1,
           msg_W2, msg_b2, pre_W1, pre_b1, pre_W2, pre_b2, ro_W1, ro_b1,
           ro_W2, ro_b2):
    n, d_in = x.shape
    ped = pe.shape[1]
    h_dim = W_embed.shape[1]
    g = 64
    out_dim = ro_W2.shape[1]

    blk = 5000
    assert n % blk == 0
    grid = (n // blk,)

    wex = W_embed[:d_in]
    wepe = W_embed[d_in:]
    batch2 = batch.reshape(n, 1)

    full = lambda shape: pl.BlockSpec(shape, lambda i: (0, 0))
    out = pl.pallas_call(
        _fused_body,
        grid=grid,
        in_specs=[
            pl.BlockSpec((blk, d_in), lambda i: (i, 0)),
            pl.BlockSpec((blk, ped), lambda i: (i, 0)),
            pl.BlockSpec((blk, 1), lambda i: (i, 0)),
            full((d_in, h_dim)),
            full((ped, h_dim)),
            full((1, h_dim)),
            full((h_dim, h_dim)),
            full((1, h_dim)),
            full((h_dim, h_dim)),
            full((1, h_dim)),
            full((h_dim, h_dim)),
            full((1, h_dim)),
            full((h_dim, out_dim)),
            full((1, out_dim)),
        ],
        out_specs=pl.BlockSpec((g, out_dim), lambda i: (0, 0)),
        out_shape=jax.ShapeDtypeStruct((g, out_dim), jnp.float32),
        scratch_shapes=[pltpu.VMEM((g, h_dim), jnp.float32)],
        compiler_params=pltpu.CompilerParams(
            dimension_semantics=("arbitrary",)),
    )(x, pe, batch2, wex, wepe, b_embed.reshape(1, h_dim),
      pre_W1, pre_b1.reshape(1, h_dim), pre_W2, pre_b2.reshape(1, h_dim),
      ro_W1, ro_b1.reshape(1, h_dim), ro_W2, ro_b2.reshape(1, out_dim))
    return jnp.squeeze(out)


# speedup vs baseline: 1.1248x; 1.0205x over previous
import jax
import jax.numpy as jnp
from jax.experimental import pallas as pl
from jax.experimental.pallas import tpu as pltpu


def _fused_body(x_ref, pe_ref, batch_ref, wex_ref, wepe_ref, be_ref,
                pw1_ref, pb1_ref, pw2_ref, pb2_ref,
                rw1_ref, rb1_ref, rw2_ref, rb2_ref,
                out_ref, acc_ref):
    i = pl.program_id(0)
    nblk = pl.num_programs(0)

    @pl.when(i == 0)
    def _init():
        acc_ref[:] = jnp.zeros_like(acc_ref)

    h = jnp.dot(x_ref[:], wex_ref[:], preferred_element_type=jnp.float32)
    h += jnp.dot(pe_ref[:], wepe_ref[:], preferred_element_type=jnp.float32)
    h = (h + be_ref[:]) * 16.0

    t = jax.nn.silu(
        jnp.dot(h, pw1_ref[:], preferred_element_type=jnp.float32) + pb1_ref[:])
    t = jnp.dot(t, pw2_ref[:], preferred_element_type=jnp.float32) + pb2_ref[:]

    g = acc_ref.shape[0]
    onehot = (batch_ref[:] == jax.lax.broadcasted_iota(
        jnp.int32, (t.shape[0], g), 1)).astype(jnp.float32)
    acc_ref[:] += jax.lax.dot_general(
        onehot, t, (((0,), (0,)), ((), ())),
        preferred_element_type=jnp.float32)

    @pl.when(i == nblk - 1)
    def _readout():
        p = acc_ref[:]
        r = jax.nn.silu(
            jnp.dot(p, rw1_ref[:], preferred_element_type=jnp.float32)
            + rb1_ref[:])
        out_ref[:] = (
            jnp.dot(r, rw2_ref[:], preferred_element_type=jnp.float32)
            + rb2_ref[:])


def kernel(x, pos, pe, edge_index, batch, W_embed, b_embed, msg_W1, msg_b1,
           msg_W2, msg_b2, pre_W1, pre_b1, pre_W2, pre_b2, ro_W1, ro_b1,
           ro_W2, ro_b2):
    n, d_in = x.shape
    ped = pe.shape[1]
    h_dim = W_embed.shape[1]
    g = 64
    out_dim = ro_W2.shape[1]

    blk = 2000
    assert n % blk == 0
    grid = (n // blk,)

    wex = W_embed[:d_in]
    wepe = W_embed[d_in:]
    batch2 = batch.reshape(n, 1)

    full = lambda shape: pl.BlockSpec(shape, lambda i: (0, 0))
    out = pl.pallas_call(
        _fused_body,
        grid=grid,
        in_specs=[
            pl.BlockSpec((blk, d_in), lambda i: (i, 0)),
            pl.BlockSpec((blk, ped), lambda i: (i, 0)),
            pl.BlockSpec((blk, 1), lambda i: (i, 0)),
            full((d_in, h_dim)),
            full((ped, h_dim)),
            full((1, h_dim)),
            full((h_dim, h_dim)),
            full((1, h_dim)),
            full((h_dim, h_dim)),
            full((1, h_dim)),
            full((h_dim, h_dim)),
            full((1, h_dim)),
            full((h_dim, out_dim)),
            full((1, out_dim)),
        ],
        out_specs=pl.BlockSpec((g, out_dim), lambda i: (0, 0)),
        out_shape=jax.ShapeDtypeStruct((g, out_dim), jnp.float32),
        scratch_shapes=[pltpu.VMEM((g, h_dim), jnp.float32)],
        compiler_params=pltpu.CompilerParams(
            dimension_semantics=("arbitrary",)),
    )(x, pe, batch2, wex, wepe, b_embed.reshape(1, h_dim),
      pre_W1, pre_b1.reshape(1, h_dim), pre_W2, pre_b2.reshape(1, h_dim),
      ro_W1, ro_b1.reshape(1, h_dim), ro_W2, ro_b2.reshape(1, out_dim))
    return jnp.squeeze(out)


# speedup vs baseline: 1.2703x; 1.1293x over previous
import jax
import jax.numpy as jnp
from jax.experimental import pallas as pl
from jax.experimental.pallas import tpu as pltpu


def _fused_body(x_ref, pe_ref, batch_ref, we_ref, be_ref,
                pw1_ref, pb1_ref, pw2_ref, pb2_ref,
                rw1_ref, rb1_ref, rw2_ref, rb2_ref,
                out_ref, acc_ref):
    i = pl.program_id(0)
    nblk = pl.num_programs(0)

    @pl.when(i == 0)
    def _init():
        acc_ref[:] = jnp.zeros_like(acc_ref)

    d_in = x_ref.shape[1]
    h = jnp.dot(x_ref[:], we_ref[:d_in, :], preferred_element_type=jnp.float32)
    h += jnp.dot(pe_ref[:], we_ref[d_in:, :], preferred_element_type=jnp.float32)
    h = (h + be_ref[:]) * 16.0

    t = jax.nn.silu(
        jnp.dot(h, pw1_ref[:], preferred_element_type=jnp.float32) + pb1_ref[:])
    t = jnp.dot(t, pw2_ref[:], preferred_element_type=jnp.float32) + pb2_ref[:]

    g = acc_ref.shape[0]
    onehot = (batch_ref[:] == jax.lax.broadcasted_iota(
        jnp.int32, (t.shape[0], g), 1)).astype(jnp.float32)
    acc_ref[:] += jax.lax.dot_general(
        onehot, t, (((0,), (0,)), ((), ())),
        preferred_element_type=jnp.float32)

    @pl.when(i == nblk - 1)
    def _readout():
        p = acc_ref[:]
        r = jax.nn.silu(
            jnp.dot(p, rw1_ref[:], preferred_element_type=jnp.float32)
            + rb1_ref[:])
        out_ref[:] = (
            jnp.dot(r, rw2_ref[:], preferred_element_type=jnp.float32)
            + rb2_ref[:])


def kernel(x, pos, pe, edge_index, batch, W_embed, b_embed, msg_W1, msg_b1,
           msg_W2, msg_b2, pre_W1, pre_b1, pre_W2, pre_b2, ro_W1, ro_b1,
           ro_W2, ro_b2):
    n, d_in = x.shape
    ped = pe.shape[1]
    h_dim = W_embed.shape[1]
    g = 64
    out_dim = ro_W2.shape[1]

    blk = 2000
    assert n % blk == 0
    grid = (n // blk,)

    batch2 = batch.reshape(n, 1)

    full = lambda shape: pl.BlockSpec(shape, lambda i: (0, 0))
    out = pl.pallas_call(
        _fused_body,
        grid=grid,
        in_specs=[
            pl.BlockSpec((blk, d_in), lambda i: (i, 0)),
            pl.BlockSpec((blk, ped), lambda i: (i, 0)),
            pl.BlockSpec((blk, 1), lambda i: (i, 0)),
            full((d_in + ped, h_dim)),
            full((1, h_dim)),
            full((h_dim, h_dim)),
            full((1, h_dim)),
            full((h_dim, h_dim)),
            full((1, h_dim)),
            full((h_dim, h_dim)),
            full((1, h_dim)),
            full((h_dim, out_dim)),
            full((1, out_dim)),
        ],
        out_specs=pl.BlockSpec((g, out_dim), lambda i: (0, 0)),
        out_shape=jax.ShapeDtypeStruct((g, out_dim), jnp.float32),
        scratch_shapes=[pltpu.VMEM((g, h_dim), jnp.float32)],
        compiler_params=pltpu.CompilerParams(
            dimension_semantics=("arbitrary",)),
    )(x, pe, batch2, W_embed, b_embed.reshape(1, h_dim),
      pre_W1, pre_b1.reshape(1, h_dim), pre_W2, pre_b2.reshape(1, h_dim),
      ro_W1, ro_b1.reshape(1, h_dim), ro_W2, ro_b2.reshape(1, out_dim))
    return jnp.squeeze(out)
